# baseline (speedup 1.0000x reference)
import functools

import jax
import jax.numpy as jnp
from jax.experimental import pallas as pl
from jax.experimental.pallas import tpu as pltpu

_ALPHA = 200.0
_BLK = 1024
_NROW = 2048
_NCOL = 2048


def _voting_kernel(nrow_ref, s_ref, o_ref):
    b = pl.program_id(0)
    i = pl.program_id(1)
    n = nrow_ref[b]
    row0 = i * _BLK

    @pl.when(row0 >= n)
    def _():
        o_ref[...] = jnp.zeros_like(o_ref)

    @pl.when(row0 < n)
    def _():
        x = s_ref[0] * _ALPHA
        m = jnp.max(x, axis=-1, keepdims=True)
        e = jnp.exp(x - m)
        sm = e / jnp.sum(e, axis=-1, keepdims=True)
        row = row0 + jax.lax.broadcasted_iota(jnp.int32, (_BLK, _NCOL), 0)
        o_ref[0] = jnp.where(row < n, sm, 0.0)


def _s_index_map(b, i, nrow_ref):
    n = nrow_ref[b]
    last_valid = jnp.maximum(pl.cdiv(n, _BLK) - 1, 0)
    return b, jnp.minimum(i, last_valid), 0


@jax.jit
def kernel(s, nrow_gt):
    grid_spec = pltpu.PrefetchScalarGridSpec(
        num_scalar_prefetch=1,
        grid=(s.shape[0], _NROW // _BLK),
        in_specs=[
            pl.BlockSpec((1, _BLK, _NCOL), _s_index_map),
        ],
        out_specs=pl.BlockSpec((1, _BLK, _NCOL), lambda b, i, n_ref: (b, i, 0)),
    )
    return pl.pallas_call(
        _voting_kernel,
        grid_spec=grid_spec,
        out_shape=jax.ShapeDtypeStruct(s.shape, s.dtype),
    )(nrow_gt, s)


# speedup vs baseline: 1.1777x; 1.1777x over previous
import functools

import jax
import jax.numpy as jnp
from jax.experimental import pallas as pl
from jax.experimental.pallas import tpu as pltpu

_ALPHA = 200.0
_BLK = 512
_NROW = 2048
_NCOL = 2048
_NB = _NROW // _BLK
_NBATCH = 8
_NBLOCKS = _NBATCH * _NB


def _softmax_block(x, rows_valid):
    x = x * _ALPHA
    m = jnp.max(x, axis=-1, keepdims=True)
    e = jnp.exp(x - m)
    r = 1.0 / jnp.sum(e, axis=-1, keepdims=True)
    sm = e * r
    row = jax.lax.broadcasted_iota(jnp.int32, x.shape, 0)
    return jnp.where(row < rows_valid, sm, 0.0)


def _voting_kernel(nrow_ref, s_hbm, o_hbm, inb, outb, zb, insems, outsems,
                   zsem, bof, iof):
    zb[...] = jnp.zeros_like(zb)

    def zero_tail(b, nz):
        nv = pl.cdiv(nrow_ref[b], _BLK)

        def start_zero(i, nz):
            pltpu.make_async_copy(
                zb, o_hbm.at[b, pl.ds(i * _BLK, _BLK), :], zsem).start()
            return nz + 1

        return jax.lax.fori_loop(nv, _NB, start_zero, nz)

    nz = jax.lax.fori_loop(0, _NBATCH, zero_tail, 0)

    def collect(g, k):
        b = g // _NB
        i = g % _NB
        valid = i * _BLK < nrow_ref[b]

        @pl.when(valid)
        def _():
            bof[k] = b
            iof[k] = i

        return k + jnp.where(valid, 1, 0)

    kv = jax.lax.fori_loop(0, _NBLOCKS, collect, 0)

    def in_copy(k, slot):
        b = bof[k]
        i = iof[k]
        return pltpu.make_async_copy(
            s_hbm.at[b, pl.ds(i * _BLK, _BLK), :], inb.at[slot],
            insems.at[slot])

    def out_copy(k, slot):
        b = bof[k]
        i = iof[k]
        return pltpu.make_async_copy(
            outb.at[slot], o_hbm.at[b, pl.ds(i * _BLK, _BLK), :],
            outsems.at[slot])

    @pl.when(kv > 0)
    def _():
        in_copy(0, 0).start()

    def step(k, carry):
        slot = jax.lax.rem(k, 2)

        @pl.when(k + 1 < kv)
        def _():
            in_copy(k + 1, 1 - slot).start()

        in_copy(k, slot).wait()

        @pl.when(k >= 2)
        def _():
            out_copy(k - 2, slot).wait()

        rows_valid = nrow_ref[bof[k]] - iof[k] * _BLK
        outb[slot] = _softmax_block(inb[slot], rows_valid)
        out_copy(k, slot).start()
        return carry

    jax.lax.fori_loop(0, kv, step, 0)

    def drain_out(k, carry):
        out_copy(k, jax.lax.rem(k, 2)).wait()
        return carry

    jax.lax.fori_loop(jnp.maximum(kv - 2, 0), kv, drain_out, 0)

    def drain_zero(j, carry):
        pltpu.make_async_copy(
            zb, o_hbm.at[0, pl.ds(0, _BLK), :], zsem).wait()
        return carry

    jax.lax.fori_loop(0, nz, drain_zero, 0)


@jax.jit
def kernel(s, nrow_gt):
    grid_spec = pltpu.PrefetchScalarGridSpec(
        num_scalar_prefetch=1,
        grid=(1,),
        in_specs=[pl.BlockSpec(memory_space=pl.ANY)],
        out_specs=pl.BlockSpec(memory_space=pl.ANY),
        scratch_shapes=[
            pltpu.VMEM((2, _BLK, _NCOL), jnp.float32),
            pltpu.VMEM((2, _BLK, _NCOL), jnp.float32),
            pltpu.VMEM((_BLK, _NCOL), jnp.float32),
            pltpu.SemaphoreType.DMA((2,)),
            pltpu.SemaphoreType.DMA((2,)),
            pltpu.SemaphoreType.DMA,
            pltpu.SMEM((_NBLOCKS + 1,), jnp.int32),
            pltpu.SMEM((_NBLOCKS + 1,), jnp.int32),
        ],
    )
    return pl.pallas_call(
        _voting_kernel,
        grid_spec=grid_spec,
        out_shape=jax.ShapeDtypeStruct(s.shape, s.dtype),
    )(nrow_gt, s)
